# Initial kernel scaffold; baseline (speedup 1.0000x reference)
import jax
import jax.numpy as jnp
from jax.experimental import pallas as pl


def kernel(x, grid, lower, resolution):
    raise NotImplementedError("write your pallas kernel here")


# speedup vs baseline: 1.1956x; 1.1956x over previous
import functools

import jax
import jax.numpy as jnp
from jax import lax
from jax.experimental import pallas as pl
from jax.experimental.pallas import tpu as pltpu
from jax.experimental.pallas import tpu_sc as plsc

_NC, _NS, _L = 2, 16, 16
_NW = _NC * _NS

_C = 2048
_G = _C // _L
_NROW = 8 * (_C // 128)


def _tec_body(nchunk, dims, x_hbm, grid_hbm, par_hbm, out_hbm,
              x0v, x1v, x2v, pv, idxv, valsv, fxv, fyv, fzv, vfv, outv,
              gsem):
    gx, gy, gz = dims
    sx, sy = gy * gz, gz
    offs = (0, 1, sy, sy + 1, sx, sx + 1, sx + sy, sx + sy + 1)
    wid = lax.axis_index("s") * _NC + lax.axis_index("c")
    ppw = nchunk * _C

    pltpu.sync_copy(par_hbm, pv)

    @pl.loop(0, nchunk)
    def chunk_loop(ci):
        base = wid * ppw + ci * _C
        pltpu.sync_copy(x_hbm.at[pl.ds(base, _C)], x0v)
        pltpu.sync_copy(x_hbm.at[pl.ds(base + ppw * _NW, _C)], x1v)
        pltpu.sync_copy(x_hbm.at[pl.ds(base + 2 * ppw * _NW, _C)], x2v)

        lxv = pv[pl.ds(0, _L)]
        lyv = pv[pl.ds(_L, _L)]
        lzv = pv[pl.ds(2 * _L, _L)]
        resv = pv[pl.ds(3 * _L, _L)]

        @pl.loop(0, _G)
        def compute(j):
            p0 = j * _L
            px = x0v[pl.ds(p0, _L)]
            py = x1v[pl.ds(p0, _L)]
            pz = x2v[pl.ds(p0, _L)]
            ix = (px - lxv) / resv
            iy = (py - lyv) / resv
            iz = (pz - lzv) / resv
            valid = ((ix >= 0.0) & (ix <= gx - 1.0)
                     & (iy >= 0.0) & (iy <= gy - 1.0)
                     & (iz >= 0.0) & (iz <= gz - 1.0))
            x0 = jnp.clip(ix.astype(jnp.int32), 0, gx - 2)
            y0 = jnp.clip(iy.astype(jnp.int32), 0, gy - 2)
            z0 = jnp.clip(iz.astype(jnp.int32), 0, gz - 2)
            fx = ix - x0.astype(jnp.float32)
            fy = iy - y0.astype(jnp.float32)
            fz = iz - z0.astype(jnp.float32)
            vf = jnp.where(valid, jnp.float32(1.0), jnp.float32(0.0))
            flat = x0 * sx + y0 * sy + z0
            for k in range(8):
                idxv[pl.ds(k * _C + p0, _L)] = flat + offs[k]
            fxv[pl.ds(p0, _L)] = fx
            fyv[pl.ds(p0, _L)] = fy
            fzv[pl.ds(p0, _L)] = fz
            vfv[pl.ds(p0, _L)] = vf

        @pl.loop(0, _NROW)
        def fire(m):
            pltpu.async_copy(grid_hbm.at[idxv.at[pl.ds(m * 128, 128)]],
                             valsv.at[pl.ds(m * 128, 128)], gsem)

        @pl.loop(0, _NROW)
        def drain(m):
            pltpu.make_async_copy(grid_hbm.at[idxv.at[pl.ds(m * 128, 128)]],
                                  valsv.at[pl.ds(m * 128, 128)], gsem).wait()

        @pl.loop(0, _G)
        def interp(j):
            p0 = j * _L
            c = [valsv[pl.ds(k * _C + p0, _L)] for k in range(8)]
            fx = fxv[pl.ds(p0, _L)]
            fy = fyv[pl.ds(p0, _L)]
            fz = fzv[pl.ds(p0, _L)]
            vf = vfv[pl.ds(p0, _L)]
            c00 = c[0] * (1 - fz) + c[1] * fz
            c01 = c[2] * (1 - fz) + c[3] * fz
            c10 = c[4] * (1 - fz) + c[5] * fz
            c11 = c[6] * (1 - fz) + c[7] * fz
            c0 = c00 * (1 - fy) + c01 * fy
            c1 = c10 * (1 - fy) + c11 * fy
            outv[pl.ds(p0, _L)] = (c0 * (1 - fx) + c1 * fx) * vf

        pltpu.sync_copy(outv, out_hbm.at[pl.ds(base, _C)])


def kernel(x, grid, lower, resolution):
    n = x.shape[0]
    dims = grid.shape
    assert n % (_NW * _C) == 0
    nchunk = n // (_NW * _C)

    params = jnp.concatenate([
        jnp.full((_L,), lower[0], jnp.float32),
        jnp.full((_L,), lower[1], jnp.float32),
        jnp.full((_L,), lower[2], jnp.float32),
        jnp.full((_L,), resolution, jnp.float32),
    ])

    mesh = plsc.VectorSubcoreMesh(core_axis_name="c", subcore_axis_name="s",
                                  num_cores=_NC, num_subcores=_NS)
    f = pl.kernel(
        functools.partial(_tec_body, nchunk, dims),
        out_type=jax.ShapeDtypeStruct((n,), jnp.float32),
        mesh=mesh,
        scratch_types=[
            pltpu.VMEM((_C,), jnp.float32),
            pltpu.VMEM((_C,), jnp.float32),
            pltpu.VMEM((_C,), jnp.float32),
            pltpu.VMEM((4 * _L,), jnp.float32),
            pltpu.VMEM((8 * _C,), jnp.int32),
            pltpu.VMEM((8 * _C,), jnp.float32),
            pltpu.VMEM((_C,), jnp.float32),
            pltpu.VMEM((_C,), jnp.float32),
            pltpu.VMEM((_C,), jnp.float32),
            pltpu.VMEM((_C,), jnp.float32),
            pltpu.VMEM((_C,), jnp.float32),
            pltpu.SemaphoreType.DMA,
        ],
    )
    return f(x.T.reshape(-1), grid.reshape(-1), params)


# speedup vs baseline: 1.3781x; 1.1527x over previous
import functools

import jax
import jax.numpy as jnp
from jax import lax
from jax.experimental import pallas as pl
from jax.experimental.pallas import tpu as pltpu
from jax.experimental.pallas import tpu_sc as plsc

_NC, _NS, _L = 2, 16, 16
_NW = _NC * _NS

_C = 2048
_G = _C // _L
_NROW = 8 * (_C // 128)


def _tec_body(nchunk, dims, x_hbm, grid_hbm, par_hbm, out_hbm,
              x0v, x1v, x2v, pv, idx0, idx1, vals0, vals1, w0, w1, outv,
              sem0, sem1):
    gx, gy, gz = dims
    sx, sy = gy * gz, gz
    offs = (0, 1, sy, sy + 1, sx, sx + 1, sx + sy, sx + sy + 1)
    wid = lax.axis_index("s") * _NC + lax.axis_index("c")
    ppw = nchunk * _C
    n_total = ppw * _NW

    pltpu.sync_copy(par_hbm, pv)

    def front(ci, idxv, wv, sem):
        base = wid * ppw + ci * _C
        pltpu.sync_copy(x_hbm.at[pl.ds(base, _C)], x0v)
        pltpu.sync_copy(x_hbm.at[pl.ds(base + n_total, _C)], x1v)
        pltpu.sync_copy(x_hbm.at[pl.ds(base + 2 * n_total, _C)], x2v)

        lxv = pv[pl.ds(0, _L)]
        lyv = pv[pl.ds(_L, _L)]
        lzv = pv[pl.ds(2 * _L, _L)]
        resv = pv[pl.ds(3 * _L, _L)]

        @pl.loop(0, _G)
        def compute(j):
            p0 = j * _L
            px = x0v[pl.ds(p0, _L)]
            py = x1v[pl.ds(p0, _L)]
            pz = x2v[pl.ds(p0, _L)]
            ix = (px - lxv) / resv
            iy = (py - lyv) / resv
            iz = (pz - lzv) / resv
            valid = ((ix >= 0.0) & (ix <= gx - 1.0)
                     & (iy >= 0.0) & (iy <= gy - 1.0)
                     & (iz >= 0.0) & (iz <= gz - 1.0))
            x0 = jnp.clip(ix.astype(jnp.int32), 0, gx - 2)
            y0 = jnp.clip(iy.astype(jnp.int32), 0, gy - 2)
            z0 = jnp.clip(iz.astype(jnp.int32), 0, gz - 2)
            fx = ix - x0.astype(jnp.float32)
            fy = iy - y0.astype(jnp.float32)
            fz = iz - z0.astype(jnp.float32)
            vf = jnp.where(valid, jnp.float32(1.0), jnp.float32(0.0))
            flat = x0 * sx + y0 * sy + z0
            for k in range(8):
                idxv[pl.ds(k * _C + p0, _L)] = flat + offs[k]
            wv[pl.ds(0 * _C + p0, _L)] = fx
            wv[pl.ds(1 * _C + p0, _L)] = fy
            wv[pl.ds(2 * _C + p0, _L)] = fz
            wv[pl.ds(3 * _C + p0, _L)] = vf

        valsv = vals0 if sem is sem0 else vals1

        @pl.loop(0, _NROW)
        def fire(m):
            pltpu.async_copy(grid_hbm.at[idxv.at[pl.ds(m * 128, 128)]],
                             valsv.at[pl.ds(m * 128, 128)], sem)

    def back(ci, idxv, valsv, wv, sem):
        base = wid * ppw + ci * _C

        @pl.loop(0, _NROW)
        def drain(m):
            pltpu.make_async_copy(grid_hbm.at[idxv.at[pl.ds(m * 128, 128)]],
                                  valsv.at[pl.ds(m * 128, 128)], sem).wait()

        @pl.loop(0, _G)
        def interp(j):
            p0 = j * _L
            c = [valsv[pl.ds(k * _C + p0, _L)] for k in range(8)]
            fx = wv[pl.ds(0 * _C + p0, _L)]
            fy = wv[pl.ds(1 * _C + p0, _L)]
            fz = wv[pl.ds(2 * _C + p0, _L)]
            vf = wv[pl.ds(3 * _C + p0, _L)]
            c00 = c[0] * (1 - fz) + c[1] * fz
            c01 = c[2] * (1 - fz) + c[3] * fz
            c10 = c[4] * (1 - fz) + c[5] * fz
            c11 = c[6] * (1 - fz) + c[7] * fz
            c0 = c00 * (1 - fy) + c01 * fy
            c1 = c10 * (1 - fy) + c11 * fy
            outv[pl.ds(p0, _L)] = (c0 * (1 - fx) + c1 * fx) * vf

        pltpu.sync_copy(outv, out_hbm.at[pl.ds(base, _C)])

    front(0, idx0, w0, sem0)

    @pl.loop(0, nchunk, step=2)
    def chunk_loop(ci):
        front(ci + 1, idx1, w1, sem1)
        back(ci, idx0, vals0, w0, sem0)

        @pl.when(ci + 2 < nchunk)
        def _():
            front(ci + 2, idx0, w0, sem0)

        back(ci + 1, idx1, vals1, w1, sem1)


def kernel(x, grid, lower, resolution):
    n = x.shape[0]
    dims = grid.shape
    assert n % (_NW * _C) == 0
    nchunk = n // (_NW * _C)
    assert nchunk % 2 == 0

    params = jnp.concatenate([
        jnp.full((_L,), lower[0], jnp.float32),
        jnp.full((_L,), lower[1], jnp.float32),
        jnp.full((_L,), lower[2], jnp.float32),
        jnp.full((_L,), resolution, jnp.float32),
    ])

    mesh = plsc.VectorSubcoreMesh(core_axis_name="c", subcore_axis_name="s",
                                  num_cores=_NC, num_subcores=_NS)
    f = pl.kernel(
        functools.partial(_tec_body, nchunk, dims),
        out_type=jax.ShapeDtypeStruct((n,), jnp.float32),
        mesh=mesh,
        scratch_types=[
            pltpu.VMEM((_C,), jnp.float32),
            pltpu.VMEM((_C,), jnp.float32),
            pltpu.VMEM((_C,), jnp.float32),
            pltpu.VMEM((4 * _L,), jnp.float32),
            pltpu.VMEM((8 * _C,), jnp.int32),
            pltpu.VMEM((8 * _C,), jnp.int32),
            pltpu.VMEM((8 * _C,), jnp.float32),
            pltpu.VMEM((8 * _C,), jnp.float32),
            pltpu.VMEM((4 * _C,), jnp.float32),
            pltpu.VMEM((4 * _C,), jnp.float32),
            pltpu.VMEM((_C,), jnp.float32),
            pltpu.SemaphoreType.DMA,
            pltpu.SemaphoreType.DMA,
        ],
    )
    return f(x.T.reshape(-1), grid.reshape(-1), params)


# speedup vs baseline: 1.4382x; 1.0436x over previous
import functools

import jax
import jax.numpy as jnp
from jax import lax
from jax.experimental import pallas as pl
from jax.experimental.pallas import tpu as pltpu
from jax.experimental.pallas import tpu_sc as plsc

_NC, _NS, _L = 2, 16, 16
_NW = _NC * _NS

_C = 2048
_G = _C // _L
_NROW = 8 * (_C // 128)


def _tec_body(nchunk, dims, x_hbm, grid_hbm, par_hbm, out_hbm,
              x0v, x1v, x2v, pv, idx0, idx1, vals0, vals1, w0, w1, outv,
              sem0, sem1):
    gx, gy, gz = dims
    sx, sy = gy * gz, gz
    offs = (0, 1, sy, sy + 1, sx, sx + 1, sx + sy, sx + sy + 1)
    wid = lax.axis_index("s") * _NC + lax.axis_index("c")
    ppw = nchunk * _C
    n_total = ppw * _NW

    pltpu.sync_copy(par_hbm, pv)

    def front(ci, idxv, wv, sem):
        base = wid * ppw + ci * _C
        pltpu.sync_copy(x_hbm.at[pl.ds(base, _C)], x0v)
        pltpu.sync_copy(x_hbm.at[pl.ds(base + n_total, _C)], x1v)
        pltpu.sync_copy(x_hbm.at[pl.ds(base + 2 * n_total, _C)], x2v)

        lxv = pv[pl.ds(0, _L)]
        lyv = pv[pl.ds(_L, _L)]
        lzv = pv[pl.ds(2 * _L, _L)]
        resv = pv[pl.ds(3 * _L, _L)]

        @pl.loop(0, _G)
        def compute(j):
            p0 = j * _L
            px = x0v[pl.ds(p0, _L)]
            py = x1v[pl.ds(p0, _L)]
            pz = x2v[pl.ds(p0, _L)]
            ix = (px - lxv) / resv
            iy = (py - lyv) / resv
            iz = (pz - lzv) / resv
            valid = ((ix >= 0.0) & (ix <= gx - 1.0)
                     & (iy >= 0.0) & (iy <= gy - 1.0)
                     & (iz >= 0.0) & (iz <= gz - 1.0))
            x0 = jnp.clip(ix.astype(jnp.int32), 0, gx - 2)
            y0 = jnp.clip(iy.astype(jnp.int32), 0, gy - 2)
            z0 = jnp.clip(iz.astype(jnp.int32), 0, gz - 2)
            fx = ix - x0.astype(jnp.float32)
            fy = iy - y0.astype(jnp.float32)
            fz = iz - z0.astype(jnp.float32)
            vf = jnp.where(valid, jnp.float32(1.0), jnp.float32(0.0))
            flat = x0 * sx + y0 * sy + z0
            for k in range(8):
                idxv[pl.ds(k * _C + p0, _L)] = flat + offs[k]
            wv[pl.ds(0 * _C + p0, _L)] = fx
            wv[pl.ds(1 * _C + p0, _L)] = fy
            wv[pl.ds(2 * _C + p0, _L)] = fz
            wv[pl.ds(3 * _C + p0, _L)] = vf

        valsv = vals0 if sem is sem0 else vals1

        for k in range(8):
            pltpu.async_copy(grid_hbm.at[idxv.at[pl.ds(k * _C, _C)]],
                             valsv.at[pl.ds(k * _C, _C)], sem)

    def back(ci, idxv, valsv, wv, sem):
        base = wid * ppw + ci * _C

        for k in range(8):
            pltpu.make_async_copy(grid_hbm.at[idxv.at[pl.ds(k * _C, _C)]],
                                  valsv.at[pl.ds(k * _C, _C)], sem).wait()

        @pl.loop(0, _G)
        def interp(j):
            p0 = j * _L
            c = [valsv[pl.ds(k * _C + p0, _L)] for k in range(8)]
            fx = wv[pl.ds(0 * _C + p0, _L)]
            fy = wv[pl.ds(1 * _C + p0, _L)]
            fz = wv[pl.ds(2 * _C + p0, _L)]
            vf = wv[pl.ds(3 * _C + p0, _L)]
            c00 = c[0] * (1 - fz) + c[1] * fz
            c01 = c[2] * (1 - fz) + c[3] * fz
            c10 = c[4] * (1 - fz) + c[5] * fz
            c11 = c[6] * (1 - fz) + c[7] * fz
            c0 = c00 * (1 - fy) + c01 * fy
            c1 = c10 * (1 - fy) + c11 * fy
            outv[pl.ds(p0, _L)] = (c0 * (1 - fx) + c1 * fx) * vf

        pltpu.sync_copy(outv, out_hbm.at[pl.ds(base, _C)])

    front(0, idx0, w0, sem0)

    @pl.loop(0, nchunk, step=2)
    def chunk_loop(ci):
        front(ci + 1, idx1, w1, sem1)
        back(ci, idx0, vals0, w0, sem0)

        @pl.when(ci + 2 < nchunk)
        def _():
            front(ci + 2, idx0, w0, sem0)

        back(ci + 1, idx1, vals1, w1, sem1)


def kernel(x, grid, lower, resolution):
    n = x.shape[0]
    dims = grid.shape
    assert n % (_NW * _C) == 0
    nchunk = n // (_NW * _C)
    assert nchunk % 2 == 0

    params = jnp.concatenate([
        jnp.full((_L,), lower[0], jnp.float32),
        jnp.full((_L,), lower[1], jnp.float32),
        jnp.full((_L,), lower[2], jnp.float32),
        jnp.full((_L,), resolution, jnp.float32),
    ])

    mesh = plsc.VectorSubcoreMesh(core_axis_name="c", subcore_axis_name="s",
                                  num_cores=_NC, num_subcores=_NS)
    f = pl.kernel(
        functools.partial(_tec_body, nchunk, dims),
        out_type=jax.ShapeDtypeStruct((n,), jnp.float32),
        mesh=mesh,
        scratch_types=[
            pltpu.VMEM((_C,), jnp.float32),
            pltpu.VMEM((_C,), jnp.float32),
            pltpu.VMEM((_C,), jnp.float32),
            pltpu.VMEM((4 * _L,), jnp.float32),
            pltpu.VMEM((8 * _C,), jnp.int32),
            pltpu.VMEM((8 * _C,), jnp.int32),
            pltpu.VMEM((8 * _C,), jnp.float32),
            pltpu.VMEM((8 * _C,), jnp.float32),
            pltpu.VMEM((4 * _C,), jnp.float32),
            pltpu.VMEM((4 * _C,), jnp.float32),
            pltpu.VMEM((_C,), jnp.float32),
            pltpu.SemaphoreType.DMA,
            pltpu.SemaphoreType.DMA,
        ],
    )
    return f(x.T.reshape(-1), grid.reshape(-1), params)
